# baseline (speedup 1.0000x reference)
import functools
import jax
import jax.numpy as jnp
from jax import lax
from jax.experimental import pallas as pl
from jax.experimental.pallas import tpu as pltpu
from jax.experimental.pallas import tpu_sc as plsc


NC = 2
NS = 16
NW = NC * NS
R = 32
UNROLL = 8


def kernel(x, pos_emb):
    B, L, D = x.shape
    xf = x.reshape(B * L * D)
    pf = pos_emb.reshape(L * D)
    LW = L // NW
    n_chunks = LW // R
    n_steps = n_chunks * B
    CHUNK = R * D

    mesh = plsc.VectorSubcoreMesh(core_axis_name="c", subcore_axis_name="s")

    @functools.partial(
        pl.kernel,
        out_type=jax.ShapeDtypeStruct((B * L * D,), jnp.float32),
        mesh=mesh,
        scratch_types=[
            pltpu.VMEM((CHUNK,), jnp.float32),
            pltpu.VMEM((CHUNK,), jnp.float32),
            pltpu.VMEM((CHUNK,), jnp.float32),
            pltpu.SemaphoreType.DMA,
            pltpu.SemaphoreType.DMA,
            pltpu.SemaphoreType.DMA,
            pltpu.SemaphoreType.DMA,
            pltpu.SemaphoreType.DMA,
        ],
    )
    def sc_add(x_hbm, pos_hbm, out_hbm, va, vb, vp0,
               sa_in, sb_in, sa_out, sb_out, sp0):
        cid = lax.axis_index("c")
        sid = lax.axis_index("s")
        wid = sid * NC + cid
        row0 = wid * LW

        bufs = (va, vb)
        sin = (sa_in, sb_in)
        sout = (sa_out, sb_out)

        def x_off(k):
            c, b = divmod(k, B)
            return (b * L + row0 + c * R) * D

        def p_off(c):
            return (row0 + c * R) * D

        hp = pltpu.async_copy(pos_hbm.at[pl.ds(p_off(0), CHUNK)], vp0, sp0)
        hin = [None] * n_steps
        hout = [None] * n_steps
        hin[0] = pltpu.async_copy(x_hbm.at[pl.ds(x_off(0), CHUNK)], va, sa_in)

        for k in range(n_steps):
            i = k % 2
            cur = bufs[i]
            c = k // B
            pcur = vp0

            if k % B == 0 and k > 0:
                hp = pltpu.async_copy(
                    pos_hbm.at[pl.ds(p_off(c), CHUNK)], vp0, sp0)

            if k + 1 < n_steps:
                j = (k + 1) % 2
                if k >= 1:
                    hout[k - 1].wait()
                hin[k + 1] = pltpu.async_copy(
                    x_hbm.at[pl.ds(x_off(k + 1), CHUNK)], bufs[j], sin[j])

            if k % B == 0:
                hp.wait()
            hin[k].wait()

            def body(t, _):
                base = t * (16 * UNROLL)
                for u in range(UNROLL):
                    s = pl.ds(base + u * 16, 16)
                    plsc.addupdate(cur.at[s], pcur[s])
                return 0

            lax.fori_loop(0, CHUNK // (16 * UNROLL), body, 0)

            hout[k] = pltpu.async_copy(
                cur, out_hbm.at[pl.ds(x_off(k), CHUNK)], sout[i])

        hout[n_steps - 2].wait()
        hout[n_steps - 1].wait()

    out = sc_add(xf, pf)
    return out.reshape(B, L, D)


# speedup vs baseline: 4.4427x; 4.4427x over previous
import jax
import jax.numpy as jnp
from jax.experimental import pallas as pl


BLOCK_L = 2048


def _add_kernel(x_ref, pos_ref, out_ref):
    out_ref[...] = x_ref[...] + pos_ref[...]


def kernel(x, pos_emb):
    B, L, D = x.shape
    nl = L // BLOCK_L
    return pl.pallas_call(
        _add_kernel,
        grid=(nl, B),
        in_specs=[
            pl.BlockSpec((1, BLOCK_L, D), lambda l, b: (b, l, 0)),
            pl.BlockSpec((BLOCK_L, D), lambda l, b: (l, 0)),
        ],
        out_specs=pl.BlockSpec((1, BLOCK_L, D), lambda l, b: (b, l, 0)),
        out_shape=jax.ShapeDtypeStruct((B, L, D), x.dtype),
    )(x, pos_emb)


# speedup vs baseline: 4.4563x; 1.0031x over previous
import jax
from jax.experimental import pallas as pl


BLOCK_L = 2048


def _add_kernel(x_ref, pos_ref, out_ref):
    out_ref[...] = x_ref[...] + pos_ref[...]


def kernel(x, pos_emb):
    B, L, D = x.shape
    nl = L // BLOCK_L
    return pl.pallas_call(
        _add_kernel,
        grid=(nl, B),
        in_specs=[
            pl.BlockSpec((1, BLOCK_L, D), lambda l, b: (b, l, 0)),
            pl.BlockSpec((BLOCK_L, D), lambda l, b: (l, 0)),
        ],
        out_specs=pl.BlockSpec((1, BLOCK_L, D), lambda l, b: (b, l, 0)),
        out_shape=jax.ShapeDtypeStruct((B, L, D), x.dtype),
    )(x, pos_emb)
